# baseline (speedup 1.0000x reference)
import functools

import jax
import jax.numpy as jnp
from jax import lax
from jax.experimental import pallas as pl
from jax.experimental.pallas import tpu as pltpu
from jax.experimental.pallas import tpu_sc as plsc

N = 10000
NREL = 16
NEMB = 64
RT = 2 * NREL + 1
NE = 320000
NEA = 2 * NE
NT = 16384

NC, NS = 2, 16
NW = NC * NS

CW = 128
CPW = 160
NCH = NW * CPW
NEAP = NCH * CW
CPS = NCH // NS
NSEG = 2 * NREL * N
NSEGP = 352000
ACC_N = 10240

CW2 = 64
TCH_W = NT // NW // CW2

BN = 2000
NBLK = N // BN

_mesh = plsc.VectorSubcoreMesh(core_axis_name="c", subcore_axis_name="s")
_sc_params = pltpu.CompilerParams(use_tc_tiling_on_sc=False,
                                  needs_layout_passes=False)


def _t1_body(x_ref, bias_ref, w_ref, y_ref):
    x = jnp.maximum(x_ref[...] + bias_ref[...], 0.0)
    y_ref[0] = jnp.dot(x, w_ref[0], preferred_element_type=jnp.float32)


def _transform1(emb, bias, w1):
    return pl.pallas_call(
        _t1_body,
        grid=(NBLK, RT),
        in_specs=[
            pl.BlockSpec((BN, NEMB), lambda i, j: (i, 0)),
            pl.BlockSpec((1, NEMB), lambda i, j: (0, 0)),
            pl.BlockSpec((1, NEMB, NEMB), lambda i, j: (j, 0, 0)),
        ],
        out_specs=pl.BlockSpec((1, BN, NEMB), lambda i, j: (j, i, 0)),
        out_shape=jax.ShapeDtypeStruct((RT, N, NEMB), jnp.float32),
    )(emb, bias, w1)


def _t2_body(p_ref, ys_ref, bias_ref, w_ref, y_ref):
    x = p_ref[0] + p_ref[1] + ys_ref[0] + bias_ref[...]
    x = jnp.maximum(x, 0.0)
    y_ref[0] = jnp.dot(x, w_ref[0], preferred_element_type=jnp.float32)


def _transform2(parts, y_prev, bias, w2):
    return pl.pallas_call(
        _t2_body,
        grid=(NBLK, RT),
        in_specs=[
            pl.BlockSpec((NC, BN, NEMB), lambda i, j: (0, i, 0)),
            pl.BlockSpec((1, BN, NEMB), lambda i, j: (RT - 1, i, 0)),
            pl.BlockSpec((1, NEMB), lambda i, j: (0, 0)),
            pl.BlockSpec((1, NEMB, NEMB), lambda i, j: (j, 0, 0)),
        ],
        out_specs=pl.BlockSpec((1, BN, NEMB), lambda i, j: (j, i, 0)),
        out_shape=jax.ShapeDtypeStruct((RT, N, NEMB), jnp.float32),
    )(parts, y_prev, bias, w2)


def _t3_body(p_ref, ys_ref, bias_ref, rel_ref, x_ref, pen_ref):
    x_ref[...] = p_ref[0] + p_ref[1] + ys_ref[0] + bias_ref[...]

    @pl.when(pl.program_id(0) == 0)
    def _():
        r = rel_ref[...]
        pen_ref[...] = jnp.full((1, 128), jnp.sum(r * r), jnp.float32)


def _combine2(parts, y_prev, bias, relations):
    return pl.pallas_call(
        _t3_body,
        grid=(NBLK,),
        in_specs=[
            pl.BlockSpec((NC, BN, NEMB), lambda i: (0, i, 0)),
            pl.BlockSpec((1, BN, NEMB), lambda i: (RT - 1, i, 0)),
            pl.BlockSpec((1, NEMB), lambda i: (0, 0)),
            pl.BlockSpec((NREL, NEMB), lambda i: (0, 0)),
        ],
        out_specs=[
            pl.BlockSpec((BN, NEMB), lambda i: (i, 0)),
            pl.BlockSpec((1, 128), lambda i: (0, 0)),
        ],
        out_shape=[
            jax.ShapeDtypeStruct((N, NEMB), jnp.float32),
            jax.ShapeDtypeStruct((1, 128), jnp.float32),
        ],
    )(parts, y_prev, bias, relations)


@functools.partial(
    pl.kernel,
    out_type=jax.ShapeDtypeStruct((NCH, CW), jnp.float32),
    mesh=_mesh,
    compiler_params=_sc_params,
    scratch_types=[
        pltpu.VMEM_SHARED((NSEGP,), jnp.float32),
        pltpu.VMEM((2000,), jnp.float32),
        pltpu.VMEM((CW,), jnp.float32),
        pltpu.VMEM((CPS, CW), jnp.int32),
        pltpu.VMEM((CW,), jnp.float32),
        pltpu.VMEM((CW,), jnp.float32),
        pltpu.VMEM((CW,), jnp.float32),
        pltpu.VMEM((CW,), jnp.float32),
        pltpu.SemaphoreType.DMA,
        pltpu.SemaphoreType.DMA,
        pltpu.SemaphoreType.DMA,
        pltpu.SemaphoreType.DMA,
        pltpu.SemaphoreType.DMA,
    ],
)
def _counts_kernel(seg_hbm, norm_hbm, cnt_sp, zbuf, ones, segblk,
                   cbuf0, cbuf1, nbuf0, nbuf1,
                   ssem, cg0, cg1, nw0, nw1):
    cid = lax.axis_index("c")
    sid = lax.axis_index("s")
    wid = sid * NC + cid
    cbuf = (cbuf0, cbuf1)
    nbuf = (nbuf0, nbuf1)
    cg = (cg0, cg1)
    nw = (nw0, nw1)

    def zfill(i, _):
        zbuf[pl.ds(i * 16, 16)] = jnp.zeros((16,), jnp.float32)
        return 0

    lax.fori_loop(0, 2000 // 16, zfill, 0)
    for k in range(CW // 16):
        ones[pl.ds(k * 16, 16)] = jnp.ones((16,), jnp.float32)

    per_s = NSEGP // NS

    def zcnt(j, _):
        pltpu.sync_copy(zbuf, cnt_sp.at[pl.ds(sid * per_s + j * 2000, 2000)])
        return 0

    lax.fori_loop(0, per_s // 2000, zcnt, 0)
    plsc.subcore_barrier()

    pltpu.sync_copy(seg_hbm.at[pl.ds(sid * CPS, CPS), :], segblk)

    def cstep(jj, _):
        pltpu.sync_copy(ones, cnt_sp.at[segblk.at[jj]], add=True)
        return 0

    lax.fori_loop(0, CPS, cstep, 0)
    plsc.subcore_barrier()

    pltpu.sync_copy(seg_hbm.at[pl.ds(wid * CPW, CPW), :],
                    segblk.at[pl.ds(0, CPW), :])

    def _cgather(k, p):
        pltpu.async_copy(cnt_sp.at[segblk.at[k]], cbuf[p], cg[p])

    def _cgather_wait(k, p):
        pltpu.make_async_copy(cnt_sp.at[segblk.at[k]], cbuf[p],
                              cg[p]).wait()

    _cgather(0, 0)
    _cgather(1, 1)

    def nstep(m, _):
        for p in range(2):
            k = 2 * m + p
            _cgather_wait(k, p)

            @pl.when(m >= 1)
            def _():
                pltpu.make_async_copy(
                    nbuf[p], norm_hbm.at[wid * CPW + k - 2], nw[p]).wait()

            for g in range(CW // 16):
                c = cbuf[p][pl.ds(g * 16, 16)]
                nbuf[p][pl.ds(g * 16, 16)] = 1.0 / jnp.maximum(c, 1.0)

            @pl.when(m < CPW // 2 - 1)
            def _():
                _cgather(k + 2, p)

            pltpu.async_copy(nbuf[p], norm_hbm.at[wid * CPW + k], nw[p])
        return 0

    lax.fori_loop(0, CPW // 2, nstep, 0)
    for p in range(2):
        pltpu.make_async_copy(nbuf[p],
                              norm_hbm.at[wid * CPW + CPW - 2 + p],
                              nw[p]).wait()


@functools.partial(
    pl.kernel,
    out_type=jax.ShapeDtypeStruct((NC, N, NEMB), jnp.float32),
    mesh=_mesh,
    compiler_params=_sc_params,
    scratch_types=[
        pltpu.VMEM_SHARED((ACC_N, NEMB), jnp.float32),
        pltpu.VMEM((CPW, CW), jnp.int32),
        pltpu.VMEM((CPW, CW), jnp.int32),
        pltpu.VMEM((CW,), jnp.float32),
        pltpu.VMEM((CW,), jnp.float32),
        pltpu.VMEM((CW,), jnp.float32),
        pltpu.VMEM((CW,), jnp.float32),
        pltpu.VMEM((CW, NEMB), jnp.float32),
        pltpu.VMEM((CW, NEMB), jnp.float32),
        pltpu.VMEM((CW, NEMB), jnp.float32),
        pltpu.VMEM((CW, NEMB), jnp.float32),
        pltpu.SemaphoreType.DMA,
        pltpu.SemaphoreType.DMA,
        pltpu.SemaphoreType.DMA,
        pltpu.SemaphoreType.DMA,
        pltpu.SemaphoreType.DMA,
        pltpu.SemaphoreType.DMA,
        pltpu.SemaphoreType.DMA,
        pltpu.SemaphoreType.DMA,
        pltpu.SemaphoreType.DMA,
        pltpu.SemaphoreType.DMA,
        pltpu.SemaphoreType.DMA,
        pltpu.SemaphoreType.DMA,
    ],
)
def _edge_kernel(y_hbm, idx_hbm, dst_hbm, norm_hbm, out_hbm,
                 acc_sp, idxblk, dstblk,
                 nbuf0, nbuf1, nbuf2, nbuf3,
                 rows0, rows1, rows2, rows3,
                 g0, g1, g2, g3, s0, s1, s2, s3,
                 n0, n1, n2, n3):
    cid = lax.axis_index("c")
    sid = lax.axis_index("s")
    wid = sid * NC + cid
    rows = (rows0, rows1, rows2, rows3)
    nbuf = (nbuf0, nbuf1, nbuf2, nbuf3)
    gsem = (g0, g1, g2, g3)
    ssem = (s0, s1, s2, s3)
    nsem = (n0, n1, n2, n3)

    def zfill(i, _):
        r_ = i // 4
        c_ = (i % 4) * 16
        rows0[r_, pl.ds(c_, 16)] = jnp.zeros((16,), jnp.float32)
        return 0

    lax.fori_loop(0, CW * 4, zfill, 0)

    def zacc(j, _):
        pltpu.sync_copy(rows0, acc_sp.at[pl.ds(sid * 640 + j * CW, CW), :])
        return 0

    lax.fori_loop(0, 640 // CW, zacc, 0)

    base = wid * CPW
    pltpu.sync_copy(idx_hbm.at[pl.ds(base, CPW), :], idxblk)
    pltpu.sync_copy(dst_hbm.at[pl.ds(base, CPW), :], dstblk)
    plsc.subcore_barrier()

    def _nload(k, b):
        pltpu.async_copy(norm_hbm.at[base + k], nbuf[b], nsem[b])

    def _nload_wait(k, b):
        pltpu.make_async_copy(norm_hbm.at[base + k], nbuf[b],
                              nsem[b]).wait()

    def _gather(k, b):
        pltpu.async_copy(y_hbm.at[idxblk.at[k]], rows[b], gsem[b])

    def _gather_wait(k, b):
        pltpu.make_async_copy(y_hbm.at[idxblk.at[k]], rows[b],
                              gsem[b]).wait()

    def _scatter(k, b):
        pltpu.sync_copy(rows[b], acc_sp.at[dstblk.at[k]], add=True)

    _gather(0, 0)
    _gather(1, 1)
    _gather(2, 2)
    _nload(0, 0)
    _nload(1, 1)
    _nload(2, 2)

    def step(j, _):
        for t in range(4):
            k = 4 * j + t
            b = t
            _gather_wait(k, b)
            _nload_wait(k, b)
            for g in range(CW // 16):
                nv16 = nbuf[b][pl.ds(g * 16, 16)]
                for i in range(16):
                    nv = jnp.full((16,), nv16[i], jnp.float32)
                    ri = g * 16 + i
                    for q in range(NEMB // 16):
                        sl = pl.ds(q * 16, 16)
                        rows[b][ri, sl] = rows[b][ri, sl] * nv
            bn = (b + 3) % 4
            if t == 0:
                _gather(k + 3, bn)
                _nload(k + 3, bn)
            else:
                @pl.when(j < CPW // 4 - 1)
                def _():
                    _gather(k + 3, bn)
                    _nload(k + 3, bn)
            _scatter(k, b)
        return 0

    lax.fori_loop(0, CPW // 4, step, 0)
    plsc.subcore_barrier()

    @pl.when(sid < 10)
    def _():
        pltpu.sync_copy(acc_sp.at[pl.ds(sid * 1000, 1000), :],
                        out_hbm.at[cid, pl.ds(sid * 1000, 1000), :])


@functools.partial(
    pl.kernel,
    out_type=jax.ShapeDtypeStruct((NT,), jnp.float32),
    mesh=_mesh,
    compiler_params=_sc_params,
    scratch_types=[
        pltpu.VMEM((CW2,), jnp.int32),
        pltpu.VMEM((CW2,), jnp.int32),
        pltpu.VMEM((CW2,), jnp.int32),
        pltpu.VMEM((CW2, NEMB), jnp.float32),
        pltpu.VMEM((CW2, NEMB), jnp.float32),
        pltpu.VMEM((CW2, NEMB), jnp.float32),
        pltpu.VMEM((CW2,), jnp.float32),
        pltpu.SemaphoreType.DMA,
    ],
)
def _distmult_kernel(x2_hbm, rel_hbm, ts_hbm, tp_hbm, to_hbm, sc_hbm,
                     tsrow, tprow, torow, abuf, bbuf, rbuf, srow, sem):
    cid = lax.axis_index("c")
    sid = lax.axis_index("s")
    wid = sid * NC + cid

    def step(rr, _):
        base = (wid * TCH_W + rr) * CW2
        pltpu.sync_copy(ts_hbm.at[pl.ds(base, CW2)], tsrow)
        pltpu.sync_copy(tp_hbm.at[pl.ds(base, CW2)], tprow)
        pltpu.sync_copy(to_hbm.at[pl.ds(base, CW2)], torow)
        pltpu.async_copy(x2_hbm.at[tsrow], abuf, sem).wait()
        pltpu.async_copy(x2_hbm.at[torow], bbuf, sem).wait()
        pltpu.async_copy(rel_hbm.at[tprow], rbuf, sem).wait()
        for i in range(CW2):
            for k in range(NEMB // 16):
                sl = pl.ds(k * 16, 16)
                rbuf[i, sl] = abuf[i, sl] * bbuf[i, sl] * rbuf[i, sl]
        for g in range(CW2 // 16):
            ridx = g * 16 + lax.iota(jnp.int32, 16)
            acc = jnp.zeros((16,), jnp.float32)
            for d in range(NEMB):
                acc = acc + plsc.load_gather(
                    rbuf, [ridx, jnp.full((16,), d, jnp.int32)])
            srow[pl.ds(g * 16, 16)] = acc
        pltpu.sync_copy(srow, sc_hbm.at[pl.ds(base, CW2)])
        return 0

    lax.fori_loop(0, TCH_W, step, 0)


def kernel(node_embeddings, node_embeddings_bias, W1, b1, W2, b2,
           relations, graph, triples):
    s = graph[:, 0]
    r = graph[:, 1] % NREL
    o = graph[:, 2]
    npad = NEAP - NEA
    idxg = jnp.concatenate(
        [r * N + s, (r + NREL) * N + o,
         jnp.zeros((npad,), jnp.int32)]).reshape(NCH, CW)
    dst = jnp.concatenate(
        [o, s, jnp.full((npad,), N, jnp.int32)]).reshape(NCH, CW)
    seg = jnp.concatenate(
        [r * N + o, (r + NREL) * N + s,
         jnp.full((npad,), NSEG, jnp.int32)]).reshape(NCH, CW)

    norm = _counts_kernel(seg)

    bias0 = node_embeddings_bias.reshape(1, NEMB)
    y1 = _transform1(node_embeddings, bias0, W1)
    p1 = _edge_kernel(y1.reshape(RT * N, NEMB), idxg, dst, norm)
    y2 = _transform2(p1, y1, b1.reshape(1, NEMB), W2)
    p2 = _edge_kernel(y2.reshape(RT * N, NEMB), idxg, dst, norm)
    x2, pen = _combine2(p2, y2, b2.reshape(1, NEMB), relations)

    ts = triples[:, 0]
    tp = triples[:, 1] % NREL
    to = triples[:, 2]
    scores = _distmult_kernel(x2, relations, ts, tp, to)
    return scores, pen[0, 0]


# speedup vs baseline: 1.6203x; 1.6203x over previous
import functools

import jax
import jax.numpy as jnp
from jax import lax
from jax.experimental import pallas as pl
from jax.experimental.pallas import tpu as pltpu
from jax.experimental.pallas import tpu_sc as plsc

N = 10000
NREL = 16
NEMB = 64
RT = 2 * NREL + 1
NE = 320000
NEA = 2 * NE
NT = 16384

NC, NS = 2, 16
NW = NC * NS

CW = 128
CPW = 160
NCH = NW * CPW
NEAP = NCH * CW
CPS = NCH // NS
NSEG = 2 * NREL * N
NSEGP = 352000
ACC_N = 10240

CW2 = 64
TCH_W = NT // NW // CW2

BN = 2000
NBLK = N // BN

_mesh = plsc.VectorSubcoreMesh(core_axis_name="c", subcore_axis_name="s")
_sc_params = pltpu.CompilerParams(use_tc_tiling_on_sc=False,
                                  needs_layout_passes=False)


def _t1_body(x_ref, bias_ref, w_ref, y_ref):
    x = jnp.maximum(x_ref[...] + bias_ref[...], 0.0)
    y_ref[0] = jnp.dot(x, w_ref[0], preferred_element_type=jnp.float32)


def _transform1(emb, bias, w1):
    return pl.pallas_call(
        _t1_body,
        grid=(NBLK, RT),
        in_specs=[
            pl.BlockSpec((BN, NEMB), lambda i, j: (i, 0)),
            pl.BlockSpec((1, NEMB), lambda i, j: (0, 0)),
            pl.BlockSpec((1, NEMB, NEMB), lambda i, j: (j, 0, 0)),
        ],
        out_specs=pl.BlockSpec((1, BN, NEMB), lambda i, j: (j, i, 0)),
        out_shape=jax.ShapeDtypeStruct((RT, N, NEMB), jnp.float32),
    )(emb, bias, w1)


def _t2_body(p_ref, ys_ref, bias_ref, w_ref, y_ref):
    x = p_ref[0] + p_ref[1] + ys_ref[0] + bias_ref[...]
    x = jnp.maximum(x, 0.0)
    y_ref[0] = jnp.dot(x, w_ref[0], preferred_element_type=jnp.float32)


def _transform2(parts, y_prev, bias, w2):
    return pl.pallas_call(
        _t2_body,
        grid=(NBLK, RT),
        in_specs=[
            pl.BlockSpec((NC, BN, NEMB), lambda i, j: (0, i, 0)),
            pl.BlockSpec((1, BN, NEMB), lambda i, j: (RT - 1, i, 0)),
            pl.BlockSpec((1, NEMB), lambda i, j: (0, 0)),
            pl.BlockSpec((1, NEMB, NEMB), lambda i, j: (j, 0, 0)),
        ],
        out_specs=pl.BlockSpec((1, BN, NEMB), lambda i, j: (j, i, 0)),
        out_shape=jax.ShapeDtypeStruct((RT, N, NEMB), jnp.float32),
    )(parts, y_prev, bias, w2)


def _t3_body(p_ref, ys_ref, bias_ref, rel_ref, x_ref, pen_ref):
    x_ref[...] = p_ref[0] + p_ref[1] + ys_ref[0] + bias_ref[...]

    @pl.when(pl.program_id(0) == 0)
    def _():
        r = rel_ref[...]
        pen_ref[...] = jnp.full((1, 128), jnp.sum(r * r), jnp.float32)


def _combine2(parts, y_prev, bias, relations):
    return pl.pallas_call(
        _t3_body,
        grid=(NBLK,),
        in_specs=[
            pl.BlockSpec((NC, BN, NEMB), lambda i: (0, i, 0)),
            pl.BlockSpec((1, BN, NEMB), lambda i: (RT - 1, i, 0)),
            pl.BlockSpec((1, NEMB), lambda i: (0, 0)),
            pl.BlockSpec((NREL, NEMB), lambda i: (0, 0)),
        ],
        out_specs=[
            pl.BlockSpec((BN, NEMB), lambda i: (i, 0)),
            pl.BlockSpec((1, 128), lambda i: (0, 0)),
        ],
        out_shape=[
            jax.ShapeDtypeStruct((N, NEMB), jnp.float32),
            jax.ShapeDtypeStruct((1, 128), jnp.float32),
        ],
    )(parts, y_prev, bias, relations)


@functools.partial(
    pl.kernel,
    out_type=jax.ShapeDtypeStruct((NCH, CW), jnp.float32),
    mesh=_mesh,
    compiler_params=_sc_params,
    scratch_types=[
        pltpu.VMEM_SHARED((NSEGP,), jnp.float32),
        pltpu.VMEM((2000,), jnp.float32),
        pltpu.VMEM((CW,), jnp.float32),
        pltpu.VMEM((CPS, CW), jnp.int32),
        pltpu.VMEM((CW,), jnp.float32),
        pltpu.VMEM((CW,), jnp.float32),
        pltpu.VMEM((CW,), jnp.float32),
        pltpu.VMEM((CW,), jnp.float32),
        pltpu.SemaphoreType.DMA,
        pltpu.SemaphoreType.DMA,
        pltpu.SemaphoreType.DMA,
        pltpu.SemaphoreType.DMA,
        pltpu.SemaphoreType.DMA,
    ],
)
def _counts_kernel(seg_hbm, norm_hbm, cnt_sp, zbuf, ones, segblk,
                   cbuf0, cbuf1, nbuf0, nbuf1,
                   ssem, cg0, cg1, nw0, nw1):
    cid = lax.axis_index("c")
    sid = lax.axis_index("s")
    wid = sid * NC + cid
    cbuf = (cbuf0, cbuf1)
    nbuf = (nbuf0, nbuf1)
    cg = (cg0, cg1)
    nw = (nw0, nw1)

    def zfill(i, _):
        zbuf[pl.ds(i * 16, 16)] = jnp.zeros((16,), jnp.float32)
        return 0

    lax.fori_loop(0, 2000 // 16, zfill, 0)
    for k in range(CW // 16):
        ones[pl.ds(k * 16, 16)] = jnp.ones((16,), jnp.float32)

    per_s = NSEGP // NS

    def zcnt(j, _):
        pltpu.sync_copy(zbuf, cnt_sp.at[pl.ds(sid * per_s + j * 2000, 2000)])
        return 0

    lax.fori_loop(0, per_s // 2000, zcnt, 0)
    plsc.subcore_barrier()

    pltpu.sync_copy(seg_hbm.at[pl.ds(sid * CPS, CPS), :], segblk)

    def cstep(jj, _):
        pltpu.sync_copy(ones, cnt_sp.at[segblk.at[jj]], add=True)
        return 0

    lax.fori_loop(0, CPS, cstep, 0)
    plsc.subcore_barrier()

    pltpu.sync_copy(seg_hbm.at[pl.ds(wid * CPW, CPW), :],
                    segblk.at[pl.ds(0, CPW), :])

    def _cgather(k, p):
        pltpu.async_copy(cnt_sp.at[segblk.at[k]], cbuf[p], cg[p])

    def _cgather_wait(k, p):
        pltpu.make_async_copy(cnt_sp.at[segblk.at[k]], cbuf[p],
                              cg[p]).wait()

    _cgather(0, 0)
    _cgather(1, 1)

    def nstep(m, _):
        for p in range(2):
            k = 2 * m + p
            _cgather_wait(k, p)

            @pl.when(m >= 1)
            def _():
                pltpu.make_async_copy(
                    nbuf[p], norm_hbm.at[wid * CPW + k - 2], nw[p]).wait()

            for g in range(CW // 16):
                c = cbuf[p][pl.ds(g * 16, 16)]
                nbuf[p][pl.ds(g * 16, 16)] = 1.0 / jnp.maximum(c, 1.0)

            @pl.when(m < CPW // 2 - 1)
            def _():
                _cgather(k + 2, p)

            pltpu.async_copy(nbuf[p], norm_hbm.at[wid * CPW + k], nw[p])
        return 0

    lax.fori_loop(0, CPW // 2, nstep, 0)
    for p in range(2):
        pltpu.make_async_copy(nbuf[p],
                              norm_hbm.at[wid * CPW + CPW - 2 + p],
                              nw[p]).wait()


@functools.partial(
    pl.kernel,
    out_type=jax.ShapeDtypeStruct((NC, N, NEMB), jnp.float32),
    mesh=_mesh,
    compiler_params=_sc_params,
    scratch_types=[
        pltpu.VMEM_SHARED((ACC_N, NEMB), jnp.float32),
        pltpu.VMEM((CPW, CW), jnp.int32),
        pltpu.VMEM((CPW, CW), jnp.int32),
        pltpu.VMEM((CW,), jnp.float32),
        pltpu.VMEM((CW,), jnp.float32),
        pltpu.VMEM((CW,), jnp.float32),
        pltpu.VMEM((CW,), jnp.float32),
        pltpu.VMEM((CW, NEMB), jnp.float32),
        pltpu.VMEM((CW, NEMB), jnp.float32),
        pltpu.VMEM((CW, NEMB), jnp.float32),
        pltpu.VMEM((CW, NEMB), jnp.float32),
        pltpu.SemaphoreType.DMA,
        pltpu.SemaphoreType.DMA,
        pltpu.SemaphoreType.DMA,
        pltpu.SemaphoreType.DMA,
        pltpu.SemaphoreType.DMA,
        pltpu.SemaphoreType.DMA,
        pltpu.SemaphoreType.DMA,
        pltpu.SemaphoreType.DMA,
        pltpu.SemaphoreType.DMA,
        pltpu.SemaphoreType.DMA,
        pltpu.SemaphoreType.DMA,
        pltpu.SemaphoreType.DMA,
    ],
)
def _edge_kernel(y_hbm, idx_hbm, dst_hbm, norm_hbm, out_hbm,
                 acc_sp, idxblk, dstblk,
                 nbuf0, nbuf1, nbuf2, nbuf3,
                 rows0, rows1, rows2, rows3,
                 g0, g1, g2, g3, s0, s1, s2, s3,
                 n0, n1, n2, n3):
    cid = lax.axis_index("c")
    sid = lax.axis_index("s")
    wid = sid * NC + cid
    rows = (rows0, rows1, rows2, rows3)
    nbuf = (nbuf0, nbuf1, nbuf2, nbuf3)
    gsem = (g0, g1, g2, g3)
    ssem = (s0, s1, s2, s3)
    nsem = (n0, n1, n2, n3)

    def zfill(i, _):
        r_ = i // 4
        c_ = (i % 4) * 16
        rows0[r_, pl.ds(c_, 16)] = jnp.zeros((16,), jnp.float32)
        return 0

    lax.fori_loop(0, CW * 4, zfill, 0)

    def zacc(j, _):
        pltpu.sync_copy(rows0, acc_sp.at[pl.ds(sid * 640 + j * CW, CW), :])
        return 0

    lax.fori_loop(0, 640 // CW, zacc, 0)

    base = wid * CPW
    pltpu.sync_copy(idx_hbm.at[pl.ds(base, CPW), :], idxblk)
    pltpu.sync_copy(dst_hbm.at[pl.ds(base, CPW), :], dstblk)
    plsc.subcore_barrier()

    def _nload(k, b):
        pltpu.async_copy(norm_hbm.at[base + k], nbuf[b], nsem[b])

    def _nload_wait(k, b):
        pltpu.make_async_copy(norm_hbm.at[base + k], nbuf[b],
                              nsem[b]).wait()

    def _gather(k, b):
        pltpu.async_copy(y_hbm.at[idxblk.at[k]], rows[b], gsem[b])

    def _gather_wait(k, b):
        pltpu.make_async_copy(y_hbm.at[idxblk.at[k]], rows[b],
                              gsem[b]).wait()

    def _scatter(k, b):
        pltpu.sync_copy(rows[b], acc_sp.at[dstblk.at[k]], add=True)

    _gather(0, 0)
    _gather(1, 1)
    _gather(2, 2)
    _nload(0, 0)
    _nload(1, 1)
    _nload(2, 2)

    def step(j, _):
        for t in range(4):
            k = 4 * j + t
            b = t
            _gather_wait(k, b)
            _nload_wait(k, b)
            for g in range(CW // 16):
                nv16 = nbuf[b][pl.ds(g * 16, 16)]
                for i in range(16):
                    nv = jnp.full((16,), nv16[i], jnp.float32)
                    ri = g * 16 + i
                    for q in range(NEMB // 16):
                        sl = pl.ds(q * 16, 16)
                        rows[b][ri, sl] = rows[b][ri, sl] * nv
            bn = (b + 3) % 4
            if t == 0:
                _gather(k + 3, bn)
                _nload(k + 3, bn)
            else:
                @pl.when(j < CPW // 4 - 1)
                def _():
                    _gather(k + 3, bn)
                    _nload(k + 3, bn)
            _scatter(k, b)
        return 0

    lax.fori_loop(0, CPW // 4, step, 0)
    plsc.subcore_barrier()

    @pl.when(sid < 10)
    def _():
        pltpu.sync_copy(acc_sp.at[pl.ds(sid * 1000, 1000), :],
                        out_hbm.at[cid, pl.ds(sid * 1000, 1000), :])


@functools.partial(
    pl.kernel,
    out_type=jax.ShapeDtypeStruct((NT,), jnp.float32),
    mesh=_mesh,
    compiler_params=_sc_params,
    scratch_types=[
        pltpu.VMEM((CW2,), jnp.int32),
        pltpu.VMEM((CW2,), jnp.int32),
        pltpu.VMEM((CW2,), jnp.int32),
        pltpu.VMEM((CW2, NEMB), jnp.float32),
        pltpu.VMEM((CW2, NEMB), jnp.float32),
        pltpu.VMEM((CW2, NEMB), jnp.float32),
        pltpu.VMEM((CW2,), jnp.float32),
        pltpu.SemaphoreType.DMA,
    ],
)
def _distmult_kernel(x2_hbm, rel_hbm, ts_hbm, tp_hbm, to_hbm, sc_hbm,
                     tsrow, tprow, torow, abuf, bbuf, rbuf, srow, sem):
    cid = lax.axis_index("c")
    sid = lax.axis_index("s")
    wid = sid * NC + cid

    def step(rr, _):
        base = (wid * TCH_W + rr) * CW2
        pltpu.sync_copy(ts_hbm.at[pl.ds(base, CW2)], tsrow)
        pltpu.sync_copy(tp_hbm.at[pl.ds(base, CW2)], tprow)
        pltpu.sync_copy(to_hbm.at[pl.ds(base, CW2)], torow)
        pltpu.async_copy(x2_hbm.at[tsrow], abuf, sem).wait()
        pltpu.async_copy(x2_hbm.at[torow], bbuf, sem).wait()
        pltpu.async_copy(rel_hbm.at[tprow], rbuf, sem).wait()
        for i in range(CW2):
            for k in range(NEMB // 16):
                sl = pl.ds(k * 16, 16)
                rbuf[i, sl] = abuf[i, sl] * bbuf[i, sl] * rbuf[i, sl]
        for g in range(CW2 // 16):
            ridx = g * 16 + lax.iota(jnp.int32, 16)
            acc = jnp.zeros((16,), jnp.float32)
            for d in range(NEMB):
                acc = acc + plsc.load_gather(
                    rbuf, [ridx, jnp.full((16,), d, jnp.int32)])
            srow[pl.ds(g * 16, 16)] = acc
        pltpu.sync_copy(srow, sc_hbm.at[pl.ds(base, CW2)])
        return 0

    lax.fori_loop(0, TCH_W, step, 0)


def kernel(node_embeddings, node_embeddings_bias, W1, b1, W2, b2,
           relations, graph, triples):
    s = graph[:, 0]
    r = graph[:, 1] % NREL
    o = graph[:, 2]
    npad = NEAP - NEA
    pad = jnp.arange(npad, dtype=jnp.int32)
    idxg = jnp.concatenate(
        [r * N + s, (r + NREL) * N + o, pad % NSEG]).reshape(NCH, CW)
    dst = jnp.concatenate(
        [o, s, N + pad % (ACC_N - N)]).reshape(NCH, CW)
    seg = jnp.concatenate(
        [r * N + o, (r + NREL) * N + s,
         NSEG + pad % (NSEGP - NSEG)]).reshape(NCH, CW)

    norm = _counts_kernel(seg)

    bias0 = node_embeddings_bias.reshape(1, NEMB)
    y1 = _transform1(node_embeddings, bias0, W1)
    p1 = _edge_kernel(y1.reshape(RT * N, NEMB), idxg, dst, norm)
    y2 = _transform2(p1, y1, b1.reshape(1, NEMB), W2)
    p2 = _edge_kernel(y2.reshape(RT * N, NEMB), idxg, dst, norm)
    x2, pen = _combine2(p2, y2, b2.reshape(1, NEMB), relations)

    ts = triples[:, 0]
    tp = triples[:, 1] % NREL
    to = triples[:, 2]
    scores = _distmult_kernel(x2, relations, ts, tp, to)
    return scores, pen[0, 0]


# speedup vs baseline: 1.6416x; 1.0131x over previous
import functools

import jax
import jax.numpy as jnp
from jax import lax
from jax.experimental import pallas as pl
from jax.experimental.pallas import tpu as pltpu
from jax.experimental.pallas import tpu_sc as plsc

N = 10000
NREL = 16
NEMB = 64
RT = 2 * NREL + 1
NE = 320000
NEA = 2 * NE
NT = 16384

NC, NS = 2, 16
NW = NC * NS

CW = 128
CPW = 160
NCH = NW * CPW
NEAP = NCH * CW
CPS = NCH // NS
NSEG = 2 * NREL * N
NSEGP = 352000
ACC_N = 10240

CW2 = 64
TCH_W = NT // NW // CW2

BN = 2000
NBLK = N // BN

_mesh = plsc.VectorSubcoreMesh(core_axis_name="c", subcore_axis_name="s")
_sc_params = pltpu.CompilerParams(use_tc_tiling_on_sc=False,
                                  needs_layout_passes=False)


def _t1_body(x_ref, bias_ref, w_ref, y_ref):
    x = jnp.maximum(x_ref[...] + bias_ref[...], 0.0)
    y_ref[0] = jnp.dot(x, w_ref[0], preferred_element_type=jnp.float32)


def _transform1(emb, bias, w1):
    return pl.pallas_call(
        _t1_body,
        grid=(NBLK, RT),
        in_specs=[
            pl.BlockSpec((BN, NEMB), lambda i, j: (i, 0)),
            pl.BlockSpec((1, NEMB), lambda i, j: (0, 0)),
            pl.BlockSpec((1, NEMB, NEMB), lambda i, j: (j, 0, 0)),
        ],
        out_specs=pl.BlockSpec((1, BN, NEMB), lambda i, j: (j, i, 0)),
        out_shape=jax.ShapeDtypeStruct((RT, N, NEMB), jnp.float32),
    )(emb, bias, w1)


def _t2_body(p_ref, ys_ref, bias_ref, w_ref, y_ref):
    x = p_ref[0] + p_ref[1] + ys_ref[0] + bias_ref[...]
    x = jnp.maximum(x, 0.0)
    y_ref[0] = jnp.dot(x, w_ref[0], preferred_element_type=jnp.float32)


def _transform2(parts, y_prev, bias, w2):
    return pl.pallas_call(
        _t2_body,
        grid=(NBLK, RT),
        in_specs=[
            pl.BlockSpec((NC, BN, NEMB), lambda i, j: (0, i, 0)),
            pl.BlockSpec((1, BN, NEMB), lambda i, j: (RT - 1, i, 0)),
            pl.BlockSpec((1, NEMB), lambda i, j: (0, 0)),
            pl.BlockSpec((1, NEMB, NEMB), lambda i, j: (j, 0, 0)),
        ],
        out_specs=pl.BlockSpec((1, BN, NEMB), lambda i, j: (j, i, 0)),
        out_shape=jax.ShapeDtypeStruct((RT, N, NEMB), jnp.float32),
    )(parts, y_prev, bias, w2)


def _t3_body(p_ref, ys_ref, bias_ref, rel_ref, x_ref, pen_ref):
    x_ref[...] = p_ref[0] + p_ref[1] + ys_ref[0] + bias_ref[...]

    @pl.when(pl.program_id(0) == 0)
    def _():
        r = rel_ref[...]
        pen_ref[...] = jnp.full((1, 128), jnp.sum(r * r), jnp.float32)


def _combine2(parts, y_prev, bias, relations):
    return pl.pallas_call(
        _t3_body,
        grid=(NBLK,),
        in_specs=[
            pl.BlockSpec((NC, BN, NEMB), lambda i: (0, i, 0)),
            pl.BlockSpec((1, BN, NEMB), lambda i: (RT - 1, i, 0)),
            pl.BlockSpec((1, NEMB), lambda i: (0, 0)),
            pl.BlockSpec((NREL, NEMB), lambda i: (0, 0)),
        ],
        out_specs=[
            pl.BlockSpec((BN, NEMB), lambda i: (i, 0)),
            pl.BlockSpec((1, 128), lambda i: (0, 0)),
        ],
        out_shape=[
            jax.ShapeDtypeStruct((N, NEMB), jnp.float32),
            jax.ShapeDtypeStruct((1, 128), jnp.float32),
        ],
    )(parts, y_prev, bias, relations)


@functools.partial(
    pl.kernel,
    out_type=jax.ShapeDtypeStruct((NCH, CW), jnp.float32),
    mesh=_mesh,
    compiler_params=_sc_params,
    scratch_types=[
        pltpu.VMEM_SHARED((NSEGP,), jnp.float32),
        pltpu.VMEM((2000,), jnp.float32),
        pltpu.VMEM((CW,), jnp.float32),
        pltpu.VMEM((CPS, CW), jnp.int32),
        pltpu.VMEM((CW,), jnp.float32),
        pltpu.VMEM((CW,), jnp.float32),
        pltpu.VMEM((CW,), jnp.float32),
        pltpu.VMEM((CW,), jnp.float32),
        pltpu.SemaphoreType.DMA,
        pltpu.SemaphoreType.DMA,
        pltpu.SemaphoreType.DMA,
        pltpu.SemaphoreType.DMA,
        pltpu.SemaphoreType.DMA,
    ],
)
def _counts_kernel(seg_hbm, norm_hbm, cnt_sp, zbuf, ones, segblk,
                   cbuf0, cbuf1, nbuf0, nbuf1,
                   ssem, cg0, cg1, nw0, nw1):
    cid = lax.axis_index("c")
    sid = lax.axis_index("s")
    wid = sid * NC + cid
    cbuf = (cbuf0, cbuf1)
    nbuf = (nbuf0, nbuf1)
    cg = (cg0, cg1)
    nw = (nw0, nw1)

    def zfill(i, _):
        zbuf[pl.ds(i * 16, 16)] = jnp.zeros((16,), jnp.float32)
        return 0

    lax.fori_loop(0, 2000 // 16, zfill, 0)
    for k in range(CW // 16):
        ones[pl.ds(k * 16, 16)] = jnp.ones((16,), jnp.float32)

    per_s = NSEGP // NS

    def zcnt(j, _):
        pltpu.sync_copy(zbuf, cnt_sp.at[pl.ds(sid * per_s + j * 2000, 2000)])
        return 0

    lax.fori_loop(0, per_s // 2000, zcnt, 0)
    plsc.subcore_barrier()

    pltpu.sync_copy(seg_hbm.at[pl.ds(sid * CPS, CPS), :], segblk)

    def cstep(jj, _):
        pltpu.sync_copy(ones, cnt_sp.at[segblk.at[jj]], add=True)
        return 0

    lax.fori_loop(0, CPS, cstep, 0)
    plsc.subcore_barrier()

    pltpu.sync_copy(seg_hbm.at[pl.ds(wid * CPW, CPW), :],
                    segblk.at[pl.ds(0, CPW), :])

    def _cgather(k, p):
        pltpu.async_copy(cnt_sp.at[segblk.at[k]], cbuf[p], cg[p])

    def _cgather_wait(k, p):
        pltpu.make_async_copy(cnt_sp.at[segblk.at[k]], cbuf[p],
                              cg[p]).wait()

    _cgather(0, 0)
    _cgather(1, 1)

    def nstep(m, _):
        for p in range(2):
            k = 2 * m + p
            _cgather_wait(k, p)

            @pl.when(m >= 1)
            def _():
                pltpu.make_async_copy(
                    nbuf[p], norm_hbm.at[wid * CPW + k - 2], nw[p]).wait()

            for g in range(CW // 16):
                c = cbuf[p][pl.ds(g * 16, 16)]
                nbuf[p][pl.ds(g * 16, 16)] = 1.0 / jnp.maximum(c, 1.0)

            @pl.when(m < CPW // 2 - 1)
            def _():
                _cgather(k + 2, p)

            pltpu.async_copy(nbuf[p], norm_hbm.at[wid * CPW + k], nw[p])
        return 0

    lax.fori_loop(0, CPW // 2, nstep, 0)
    for p in range(2):
        pltpu.make_async_copy(nbuf[p],
                              norm_hbm.at[wid * CPW + CPW - 2 + p],
                              nw[p]).wait()


@functools.partial(
    pl.kernel,
    out_type=jax.ShapeDtypeStruct((NC, N, NEMB), jnp.float32),
    mesh=_mesh,
    compiler_params=_sc_params,
    scratch_types=[
        pltpu.VMEM_SHARED((ACC_N, NEMB), jnp.float32),
        pltpu.VMEM((CPW, CW), jnp.int32),
        pltpu.VMEM((CPW, CW), jnp.int32),
        pltpu.VMEM((CW,), jnp.float32),
        pltpu.VMEM((CW,), jnp.float32),
        pltpu.VMEM((CW,), jnp.float32),
        pltpu.VMEM((CW,), jnp.float32),
        pltpu.VMEM((CW, NEMB), jnp.float32),
        pltpu.VMEM((CW, NEMB), jnp.float32),
        pltpu.VMEM((CW, NEMB), jnp.float32),
        pltpu.VMEM((CW, NEMB), jnp.float32),
        pltpu.SemaphoreType.DMA,
        pltpu.SemaphoreType.DMA,
        pltpu.SemaphoreType.DMA,
        pltpu.SemaphoreType.DMA,
        pltpu.SemaphoreType.DMA,
        pltpu.SemaphoreType.DMA,
        pltpu.SemaphoreType.DMA,
        pltpu.SemaphoreType.DMA,
        pltpu.SemaphoreType.DMA,
        pltpu.SemaphoreType.DMA,
        pltpu.SemaphoreType.DMA,
        pltpu.SemaphoreType.DMA,
    ],
)
def _edge_kernel(y_hbm, idx_hbm, dst_hbm, norm_hbm, out_hbm,
                 acc_sp, idxblk, dstblk,
                 nbuf0, nbuf1, nbuf2, nbuf3,
                 rows0, rows1, rows2, rows3,
                 g0, g1, g2, g3, s0, s1, s2, s3,
                 n0, n1, n2, n3):
    cid = lax.axis_index("c")
    sid = lax.axis_index("s")
    wid = sid * NC + cid
    rows = (rows0, rows1, rows2, rows3)
    nbuf = (nbuf0, nbuf1, nbuf2, nbuf3)
    gsem = (g0, g1, g2, g3)
    ssem = (s0, s1, s2, s3)
    nsem = (n0, n1, n2, n3)

    def zfill(i, _):
        r_ = i // 4
        c_ = (i % 4) * 16
        rows0[r_, pl.ds(c_, 16)] = jnp.zeros((16,), jnp.float32)
        return 0

    lax.fori_loop(0, CW * 4, zfill, 0)

    def zacc(j, _):
        pltpu.sync_copy(rows0, acc_sp.at[pl.ds(sid * 640 + j * CW, CW), :])
        return 0

    lax.fori_loop(0, 640 // CW, zacc, 0)

    base = wid * CPW
    pltpu.sync_copy(idx_hbm.at[pl.ds(base, CPW), :], idxblk)
    pltpu.sync_copy(dst_hbm.at[pl.ds(base, CPW), :], dstblk)
    plsc.subcore_barrier()

    def _nload(k, b):
        pltpu.async_copy(norm_hbm.at[base + k], nbuf[b], nsem[b])

    def _nload_wait(k, b):
        pltpu.make_async_copy(norm_hbm.at[base + k], nbuf[b],
                              nsem[b]).wait()

    def _gather(k, b):
        pltpu.async_copy(y_hbm.at[idxblk.at[k]], rows[b], gsem[b])

    def _gather_wait(k, b):
        pltpu.make_async_copy(y_hbm.at[idxblk.at[k]], rows[b],
                              gsem[b]).wait()

    def _scatter(k, b):
        pltpu.sync_copy(rows[b], acc_sp.at[dstblk.at[k]], add=True)

    _gather(0, 0)
    _gather(1, 1)
    _gather(2, 2)
    _nload(0, 0)
    _nload(1, 1)
    _nload(2, 2)

    def step(j, _):
        for t in range(4):
            k = 4 * j + t
            b = t
            _gather_wait(k, b)
            _nload_wait(k, b)
            for g in range(CW // 16):
                nv16 = nbuf[b][pl.ds(g * 16, 16)]
                for i in range(16):
                    nv = jnp.full((16,), nv16[i], jnp.float32)
                    ri = g * 16 + i
                    for q in range(NEMB // 16):
                        sl = pl.ds(q * 16, 16)
                        rows[b][ri, sl] = rows[b][ri, sl] * nv
            bn = (b + 3) % 4
            if t == 0:
                _gather(k + 3, bn)
                _nload(k + 3, bn)
            else:
                @pl.when(j < CPW // 4 - 1)
                def _():
                    _gather(k + 3, bn)
                    _nload(k + 3, bn)
            _scatter(k, b)
        return 0

    lax.fori_loop(0, CPW // 4, step, 0)
    plsc.subcore_barrier()

    @pl.when(sid < 10)
    def _():
        pltpu.sync_copy(acc_sp.at[pl.ds(sid * 1000, 1000), :],
                        out_hbm.at[cid, pl.ds(sid * 1000, 1000), :])


@functools.partial(
    pl.kernel,
    out_type=jax.ShapeDtypeStruct((NT // CW2, CW2), jnp.float32),
    mesh=_mesh,
    compiler_params=_sc_params,
    scratch_types=[
        pltpu.VMEM((TCH_W, CW2), jnp.int32),
        pltpu.VMEM((TCH_W, CW2), jnp.int32),
        pltpu.VMEM((TCH_W, CW2), jnp.int32),
        pltpu.VMEM((CW2, NEMB), jnp.float32),
        pltpu.VMEM((CW2, NEMB), jnp.float32),
        pltpu.VMEM((CW2, NEMB), jnp.float32),
        pltpu.VMEM((CW2, NEMB), jnp.float32),
        pltpu.VMEM((CW2, NEMB), jnp.float32),
        pltpu.VMEM((CW2, NEMB), jnp.float32),
        pltpu.VMEM((CW2,), jnp.float32),
        pltpu.SemaphoreType.DMA,
        pltpu.SemaphoreType.DMA,
        pltpu.SemaphoreType.DMA,
        pltpu.SemaphoreType.DMA,
        pltpu.SemaphoreType.DMA,
        pltpu.SemaphoreType.DMA,
    ],
)
def _distmult_kernel(x2_hbm, rel_hbm, ts_hbm, tp_hbm, to_hbm, sc_hbm,
                     tsblk, tpblk, toblk,
                     abuf0, bbuf0, rbuf0, abuf1, bbuf1, rbuf1, srow,
                     a0, b0, r0, a1, b1, r1):
    cid = lax.axis_index("c")
    sid = lax.axis_index("s")
    wid = sid * NC + cid
    abuf = (abuf0, abuf1)
    bbuf = (bbuf0, bbuf1)
    rbuf = (rbuf0, rbuf1)
    asem = (a0, a1)
    bsem = (b0, b1)
    rsem = (r0, r1)

    base = wid * TCH_W
    pltpu.sync_copy(ts_hbm.at[pl.ds(base, TCH_W), :], tsblk)
    pltpu.sync_copy(tp_hbm.at[pl.ds(base, TCH_W), :], tpblk)
    pltpu.sync_copy(to_hbm.at[pl.ds(base, TCH_W), :], toblk)

    def _tgather(k, p):
        pltpu.async_copy(x2_hbm.at[tsblk.at[k]], abuf[p], asem[p])
        pltpu.async_copy(x2_hbm.at[toblk.at[k]], bbuf[p], bsem[p])
        pltpu.async_copy(rel_hbm.at[tpblk.at[k]], rbuf[p], rsem[p])

    def _tgather_wait(k, p):
        pltpu.make_async_copy(x2_hbm.at[tsblk.at[k]], abuf[p],
                              asem[p]).wait()
        pltpu.make_async_copy(x2_hbm.at[toblk.at[k]], bbuf[p],
                              bsem[p]).wait()
        pltpu.make_async_copy(rel_hbm.at[tpblk.at[k]], rbuf[p],
                              rsem[p]).wait()

    _tgather(0, 0)

    def step(m, _):
        for p in range(2):
            k = 2 * m + p
            _tgather_wait(k, p)

            @pl.when(k < TCH_W - 1)
            def _():
                _tgather(k + 1, 1 - p)

            for i in range(CW2):
                for q in range(NEMB // 16):
                    sl = pl.ds(q * 16, 16)
                    rbuf[p][i, sl] = (abuf[p][i, sl] * bbuf[p][i, sl]
                                      * rbuf[p][i, sl])
            for g in range(CW2 // 16):
                ridx = g * 16 + lax.iota(jnp.int32, 16)
                acc = jnp.zeros((16,), jnp.float32)
                for d in range(NEMB):
                    acc = acc + plsc.load_gather(
                        rbuf[p], [ridx, jnp.full((16,), d, jnp.int32)])
                srow[pl.ds(g * 16, 16)] = acc
            pltpu.sync_copy(srow, sc_hbm.at[base + k])
        return 0

    lax.fori_loop(0, TCH_W // 2, step, 0)


def kernel(node_embeddings, node_embeddings_bias, W1, b1, W2, b2,
           relations, graph, triples):
    s = graph[:, 0]
    r = graph[:, 1] % NREL
    o = graph[:, 2]
    npad = NEAP - NEA
    pad = jnp.arange(npad, dtype=jnp.int32)
    idxg = jnp.concatenate(
        [r * N + s, (r + NREL) * N + o, pad % NSEG]).reshape(NCH, CW)
    dst = jnp.concatenate(
        [o, s, N + pad % (ACC_N - N)]).reshape(NCH, CW)
    seg = jnp.concatenate(
        [r * N + o, (r + NREL) * N + s,
         NSEG + pad % (NSEGP - NSEG)]).reshape(NCH, CW)

    norm = _counts_kernel(seg)

    bias0 = node_embeddings_bias.reshape(1, NEMB)
    y1 = _transform1(node_embeddings, bias0, W1)
    p1 = _edge_kernel(y1.reshape(RT * N, NEMB), idxg, dst, norm)
    y2 = _transform2(p1, y1, b1.reshape(1, NEMB), W2)
    p2 = _edge_kernel(y2.reshape(RT * N, NEMB), idxg, dst, norm)
    x2, pen = _combine2(p2, y2, b2.reshape(1, NEMB), relations)

    ts = triples[:, 0].reshape(NT // CW2, CW2)
    tp = (triples[:, 1] % NREL).reshape(NT // CW2, CW2)
    to = triples[:, 2].reshape(NT // CW2, CW2)
    scores = _distmult_kernel(x2, relations, ts, tp, to)
    return scores.reshape(NT), pen[0, 0]


# speedup vs baseline: 1.6552x; 1.0083x over previous
import functools

import jax
import jax.numpy as jnp
from jax import lax
from jax.experimental import pallas as pl
from jax.experimental.pallas import tpu as pltpu
from jax.experimental.pallas import tpu_sc as plsc

N = 10000
NREL = 16
NEMB = 64
RT = 2 * NREL + 1
NE = 320000
NEA = 2 * NE
NT = 16384

NC, NS = 2, 16
NW = NC * NS

CW = 128
CPW = 160
NCH = NW * CPW
NEAP = NCH * CW
CPS = NCH // NS
NSEG = 2 * NREL * N
NSEGP = 352000
ACC_N = 10240

CW2 = 64
TCH_W = NT // NW // CW2

BN = 2000
NBLK = N // BN

_mesh = plsc.VectorSubcoreMesh(core_axis_name="c", subcore_axis_name="s")
_sc_params = pltpu.CompilerParams(use_tc_tiling_on_sc=False,
                                  needs_layout_passes=False)


def _t1_body(x_ref, bias_ref, w_ref, y_ref):
    x = jnp.maximum(x_ref[...] + bias_ref[...], 0.0)
    y_ref[0] = jnp.dot(x, w_ref[0], preferred_element_type=jnp.float32)


def _transform1(emb, bias, w1):
    return pl.pallas_call(
        _t1_body,
        grid=(NBLK, RT),
        in_specs=[
            pl.BlockSpec((BN, NEMB), lambda i, j: (i, 0)),
            pl.BlockSpec((1, NEMB), lambda i, j: (0, 0)),
            pl.BlockSpec((1, NEMB, NEMB), lambda i, j: (j, 0, 0)),
        ],
        out_specs=pl.BlockSpec((1, BN, NEMB), lambda i, j: (j, i, 0)),
        out_shape=jax.ShapeDtypeStruct((RT, N, NEMB), jnp.float32),
    )(emb, bias, w1)


def _t2_body(p_ref, ys_ref, bias_ref, w_ref, y_ref):
    x = p_ref[0] + p_ref[1] + ys_ref[0] + bias_ref[...]
    x = jnp.maximum(x, 0.0)
    y_ref[0] = jnp.dot(x, w_ref[0], preferred_element_type=jnp.float32)


def _transform2(parts, y_prev, bias, w2):
    return pl.pallas_call(
        _t2_body,
        grid=(NBLK, RT),
        in_specs=[
            pl.BlockSpec((NC, BN, NEMB), lambda i, j: (0, i, 0)),
            pl.BlockSpec((1, BN, NEMB), lambda i, j: (RT - 1, i, 0)),
            pl.BlockSpec((1, NEMB), lambda i, j: (0, 0)),
            pl.BlockSpec((1, NEMB, NEMB), lambda i, j: (j, 0, 0)),
        ],
        out_specs=pl.BlockSpec((1, BN, NEMB), lambda i, j: (j, i, 0)),
        out_shape=jax.ShapeDtypeStruct((RT, N, NEMB), jnp.float32),
    )(parts, y_prev, bias, w2)


def _t3_body(p_ref, ys_ref, bias_ref, rel_ref, x_ref, pen_ref):
    x_ref[...] = p_ref[0] + p_ref[1] + ys_ref[0] + bias_ref[...]

    @pl.when(pl.program_id(0) == 0)
    def _():
        r = rel_ref[...]
        pen_ref[...] = jnp.full((1, 128), jnp.sum(r * r), jnp.float32)


def _combine2(parts, y_prev, bias, relations):
    return pl.pallas_call(
        _t3_body,
        grid=(NBLK,),
        in_specs=[
            pl.BlockSpec((NC, BN, NEMB), lambda i: (0, i, 0)),
            pl.BlockSpec((1, BN, NEMB), lambda i: (RT - 1, i, 0)),
            pl.BlockSpec((1, NEMB), lambda i: (0, 0)),
            pl.BlockSpec((NREL, NEMB), lambda i: (0, 0)),
        ],
        out_specs=[
            pl.BlockSpec((BN, NEMB), lambda i: (i, 0)),
            pl.BlockSpec((1, 128), lambda i: (0, 0)),
        ],
        out_shape=[
            jax.ShapeDtypeStruct((N, NEMB), jnp.float32),
            jax.ShapeDtypeStruct((1, 128), jnp.float32),
        ],
    )(parts, y_prev, bias, relations)


@functools.partial(
    pl.kernel,
    out_type=jax.ShapeDtypeStruct((NCH, CW), jnp.float32),
    mesh=_mesh,
    compiler_params=_sc_params,
    scratch_types=[
        pltpu.VMEM_SHARED((NSEGP,), jnp.float32),
        pltpu.VMEM((2000,), jnp.float32),
        pltpu.VMEM((CW,), jnp.float32),
        pltpu.VMEM((CPS, CW), jnp.int32),
        pltpu.VMEM((CW,), jnp.float32),
        pltpu.VMEM((CW,), jnp.float32),
        pltpu.VMEM((CW,), jnp.float32),
        pltpu.VMEM((CW,), jnp.float32),
        pltpu.SemaphoreType.DMA,
        pltpu.SemaphoreType.DMA,
        pltpu.SemaphoreType.DMA,
        pltpu.SemaphoreType.DMA,
        pltpu.SemaphoreType.DMA,
    ],
)
def _counts_kernel(seg_hbm, norm_hbm, cnt_sp, zbuf, ones, segblk,
                   cbuf0, cbuf1, nbuf0, nbuf1,
                   ssem, cg0, cg1, nw0, nw1):
    cid = lax.axis_index("c")
    sid = lax.axis_index("s")
    wid = sid * NC + cid
    cbuf = (cbuf0, cbuf1)
    nbuf = (nbuf0, nbuf1)
    cg = (cg0, cg1)
    nw = (nw0, nw1)

    def zfill(i, _):
        zbuf[pl.ds(i * 16, 16)] = jnp.zeros((16,), jnp.float32)
        return 0

    lax.fori_loop(0, 2000 // 16, zfill, 0)
    for k in range(CW // 16):
        ones[pl.ds(k * 16, 16)] = jnp.ones((16,), jnp.float32)

    per_s = NSEGP // NS

    def zcnt(j, _):
        pltpu.sync_copy(zbuf, cnt_sp.at[pl.ds(sid * per_s + j * 2000, 2000)])
        return 0

    lax.fori_loop(0, per_s // 2000, zcnt, 0)
    plsc.subcore_barrier()

    pltpu.sync_copy(seg_hbm.at[pl.ds(sid * CPS, CPS), :], segblk)

    def cstep(jj, _):
        pltpu.sync_copy(ones, cnt_sp.at[segblk.at[jj]], add=True)
        return 0

    lax.fori_loop(0, CPS, cstep, 0)
    plsc.subcore_barrier()

    pltpu.sync_copy(seg_hbm.at[pl.ds(wid * CPW, CPW), :],
                    segblk.at[pl.ds(0, CPW), :])

    def _cgather(k, p):
        pltpu.async_copy(cnt_sp.at[segblk.at[k]], cbuf[p], cg[p])

    def _cgather_wait(k, p):
        pltpu.make_async_copy(cnt_sp.at[segblk.at[k]], cbuf[p],
                              cg[p]).wait()

    _cgather(0, 0)
    _cgather(1, 1)

    def nstep(m, _):
        for p in range(2):
            k = 2 * m + p
            _cgather_wait(k, p)

            @pl.when(m >= 1)
            def _():
                pltpu.make_async_copy(
                    nbuf[p], norm_hbm.at[wid * CPW + k - 2], nw[p]).wait()

            for g in range(CW // 16):
                c = cbuf[p][pl.ds(g * 16, 16)]
                nbuf[p][pl.ds(g * 16, 16)] = 1.0 / jnp.maximum(c, 1.0)

            @pl.when(m < CPW // 2 - 1)
            def _():
                _cgather(k + 2, p)

            pltpu.async_copy(nbuf[p], norm_hbm.at[wid * CPW + k], nw[p])
        return 0

    lax.fori_loop(0, CPW // 2, nstep, 0)
    for p in range(2):
        pltpu.make_async_copy(nbuf[p],
                              norm_hbm.at[wid * CPW + CPW - 2 + p],
                              nw[p]).wait()


@functools.partial(
    pl.kernel,
    out_type=jax.ShapeDtypeStruct((NC, N, NEMB), jnp.float32),
    mesh=_mesh,
    compiler_params=_sc_params,
    scratch_types=[
        pltpu.VMEM_SHARED((ACC_N, NEMB), jnp.float32),
        pltpu.VMEM((CPW, CW), jnp.int32),
        pltpu.VMEM((CPW, CW), jnp.int32),
        pltpu.VMEM((CW,), jnp.float32),
        pltpu.VMEM((CW,), jnp.float32),
        pltpu.VMEM((CW,), jnp.float32),
        pltpu.VMEM((CW,), jnp.float32),
        pltpu.VMEM((CW, NEMB), jnp.float32),
        pltpu.VMEM((CW, NEMB), jnp.float32),
        pltpu.VMEM((CW, NEMB), jnp.float32),
        pltpu.VMEM((CW, NEMB), jnp.float32),
        pltpu.SemaphoreType.DMA,
        pltpu.SemaphoreType.DMA,
        pltpu.SemaphoreType.DMA,
        pltpu.SemaphoreType.DMA,
        pltpu.SemaphoreType.DMA,
        pltpu.SemaphoreType.DMA,
        pltpu.SemaphoreType.DMA,
        pltpu.SemaphoreType.DMA,
        pltpu.SemaphoreType.DMA,
        pltpu.SemaphoreType.DMA,
        pltpu.SemaphoreType.DMA,
        pltpu.SemaphoreType.DMA,
    ],
)
def _edge_kernel(y_hbm, idx_hbm, dst_hbm, norm_hbm, out_hbm,
                 acc_sp, idxblk, dstblk,
                 nbuf0, nbuf1, nbuf2, nbuf3,
                 rows0, rows1, rows2, rows3,
                 g0, g1, g2, g3, s0, s1, s2, s3,
                 n0, n1, n2, n3):
    cid = lax.axis_index("c")
    sid = lax.axis_index("s")
    wid = sid * NC + cid
    rows = (rows0, rows1, rows2, rows3)
    nbuf = (nbuf0, nbuf1, nbuf2, nbuf3)
    gsem = (g0, g1, g2, g3)
    ssem = (s0, s1, s2, s3)
    nsem = (n0, n1, n2, n3)

    def zfill(i, _):
        r_ = i // 4
        c_ = (i % 4) * 16
        rows0[r_, pl.ds(c_, 16)] = jnp.zeros((16,), jnp.float32)
        return 0

    lax.fori_loop(0, CW * 4, zfill, 0)

    def zacc(j, _):
        pltpu.sync_copy(rows0, acc_sp.at[pl.ds(sid * 640 + j * CW, CW), :])
        return 0

    lax.fori_loop(0, 640 // CW, zacc, 0)

    base = wid * CPW
    pltpu.sync_copy(idx_hbm.at[pl.ds(base, CPW), :], idxblk)
    pltpu.sync_copy(dst_hbm.at[pl.ds(base, CPW), :], dstblk)
    plsc.subcore_barrier()

    def _nload(k, b):
        pltpu.async_copy(norm_hbm.at[base + k], nbuf[b], nsem[b])

    def _nload_wait(k, b):
        pltpu.make_async_copy(norm_hbm.at[base + k], nbuf[b],
                              nsem[b]).wait()

    def _gather(k, b):
        pltpu.async_copy(y_hbm.at[idxblk.at[k]], rows[b], gsem[b])

    def _gather_wait(k, b):
        pltpu.make_async_copy(y_hbm.at[idxblk.at[k]], rows[b],
                              gsem[b]).wait()

    def _scatter(k, b):
        pltpu.sync_copy(rows[b], acc_sp.at[dstblk.at[k]], add=True)

    _gather(0, 0)
    _gather(1, 1)
    _gather(2, 2)
    _nload(0, 0)
    _nload(1, 1)
    _nload(2, 2)

    def step(j, _):
        for t in range(4):
            k = 4 * j + t
            b = t
            _gather_wait(k, b)
            _nload_wait(k, b)
            bn = (b + 3) % 4
            if t == 0:
                _gather(k + 3, bn)
                _nload(k + 3, bn)
            else:
                @pl.when(j < CPW // 4 - 1)
                def _():
                    _gather(k + 3, bn)
                    _nload(k + 3, bn)
            for g in range(CW // 16):
                nv16 = nbuf[b][pl.ds(g * 16, 16)]
                for i in range(16):
                    nv = jnp.full((16,), nv16[i], jnp.float32)
                    ri = g * 16 + i
                    for q in range(NEMB // 16):
                        sl = pl.ds(q * 16, 16)
                        rows[b][ri, sl] = rows[b][ri, sl] * nv
            _scatter(k, b)
        return 0

    lax.fori_loop(0, CPW // 4, step, 0)
    plsc.subcore_barrier()

    @pl.when(sid < 10)
    def _():
        pltpu.sync_copy(acc_sp.at[pl.ds(sid * 1000, 1000), :],
                        out_hbm.at[cid, pl.ds(sid * 1000, 1000), :])


@functools.partial(
    pl.kernel,
    out_type=jax.ShapeDtypeStruct((NT // CW2, CW2), jnp.float32),
    mesh=_mesh,
    compiler_params=_sc_params,
    scratch_types=[
        pltpu.VMEM((TCH_W, CW2), jnp.int32),
        pltpu.VMEM((TCH_W, CW2), jnp.int32),
        pltpu.VMEM((TCH_W, CW2), jnp.int32),
        pltpu.VMEM((CW2, NEMB), jnp.float32),
        pltpu.VMEM((CW2, NEMB), jnp.float32),
        pltpu.VMEM((CW2, NEMB), jnp.float32),
        pltpu.VMEM((CW2, NEMB), jnp.float32),
        pltpu.VMEM((CW2, NEMB), jnp.float32),
        pltpu.VMEM((CW2, NEMB), jnp.float32),
        pltpu.VMEM((CW2,), jnp.float32),
        pltpu.SemaphoreType.DMA,
        pltpu.SemaphoreType.DMA,
        pltpu.SemaphoreType.DMA,
        pltpu.SemaphoreType.DMA,
        pltpu.SemaphoreType.DMA,
        pltpu.SemaphoreType.DMA,
    ],
)
def _distmult_kernel(x2_hbm, rel_hbm, ts_hbm, tp_hbm, to_hbm, sc_hbm,
                     tsblk, tpblk, toblk,
                     abuf0, bbuf0, rbuf0, abuf1, bbuf1, rbuf1, srow,
                     a0, b0, r0, a1, b1, r1):
    cid = lax.axis_index("c")
    sid = lax.axis_index("s")
    wid = sid * NC + cid
    abuf = (abuf0, abuf1)
    bbuf = (bbuf0, bbuf1)
    rbuf = (rbuf0, rbuf1)
    asem = (a0, a1)
    bsem = (b0, b1)
    rsem = (r0, r1)

    base = wid * TCH_W
    pltpu.sync_copy(ts_hbm.at[pl.ds(base, TCH_W), :], tsblk)
    pltpu.sync_copy(tp_hbm.at[pl.ds(base, TCH_W), :], tpblk)
    pltpu.sync_copy(to_hbm.at[pl.ds(base, TCH_W), :], toblk)

    def _tgather(k, p):
        pltpu.async_copy(x2_hbm.at[tsblk.at[k]], abuf[p], asem[p])
        pltpu.async_copy(x2_hbm.at[toblk.at[k]], bbuf[p], bsem[p])
        pltpu.async_copy(rel_hbm.at[tpblk.at[k]], rbuf[p], rsem[p])

    def _tgather_wait(k, p):
        pltpu.make_async_copy(x2_hbm.at[tsblk.at[k]], abuf[p],
                              asem[p]).wait()
        pltpu.make_async_copy(x2_hbm.at[toblk.at[k]], bbuf[p],
                              bsem[p]).wait()
        pltpu.make_async_copy(rel_hbm.at[tpblk.at[k]], rbuf[p],
                              rsem[p]).wait()

    _tgather(0, 0)

    def step(m, _):
        for p in range(2):
            k = 2 * m + p
            _tgather_wait(k, p)

            @pl.when(k < TCH_W - 1)
            def _():
                _tgather(k + 1, 1 - p)

            for i in range(CW2):
                for q in range(NEMB // 16):
                    sl = pl.ds(q * 16, 16)
                    rbuf[p][i, sl] = (abuf[p][i, sl] * bbuf[p][i, sl]
                                      * rbuf[p][i, sl])
            for g in range(CW2 // 16):
                ridx = g * 16 + lax.iota(jnp.int32, 16)
                acc = jnp.zeros((16,), jnp.float32)
                for d in range(NEMB):
                    acc = acc + plsc.load_gather(
                        rbuf[p], [ridx, jnp.full((16,), d, jnp.int32)])
                srow[pl.ds(g * 16, 16)] = acc
            pltpu.sync_copy(srow, sc_hbm.at[base + k])
        return 0

    lax.fori_loop(0, TCH_W // 2, step, 0)


def kernel(node_embeddings, node_embeddings_bias, W1, b1, W2, b2,
           relations, graph, triples):
    s = graph[:, 0]
    r = graph[:, 1] % NREL
    o = graph[:, 2]
    npad = NEAP - NEA
    pad = jnp.arange(npad, dtype=jnp.int32)
    idxg = jnp.concatenate(
        [r * N + s, (r + NREL) * N + o, pad % NSEG]).reshape(NCH, CW)
    dst = jnp.concatenate(
        [o, s, N + pad % (ACC_N - N)]).reshape(NCH, CW)
    seg = jnp.concatenate(
        [r * N + o, (r + NREL) * N + s,
         NSEG + pad % (NSEGP - NSEG)]).reshape(NCH, CW)

    norm = _counts_kernel(seg)

    bias0 = node_embeddings_bias.reshape(1, NEMB)
    y1 = _transform1(node_embeddings, bias0, W1)
    p1 = _edge_kernel(y1.reshape(RT * N, NEMB), idxg, dst, norm)
    y2 = _transform2(p1, y1, b1.reshape(1, NEMB), W2)
    p2 = _edge_kernel(y2.reshape(RT * N, NEMB), idxg, dst, norm)
    x2, pen = _combine2(p2, y2, b2.reshape(1, NEMB), relations)

    ts = triples[:, 0].reshape(NT // CW2, CW2)
    tp = (triples[:, 1] % NREL).reshape(NT // CW2, CW2)
    to = triples[:, 2].reshape(NT // CW2, CW2)
    scores = _distmult_kernel(x2, relations, ts, tp, to)
    return scores.reshape(NT), pen[0, 0]
